# Initial kernel scaffold; baseline (speedup 1.0000x reference)
import functools

import jax
import jax.numpy as jnp
from jax import lax
from jax.experimental import pallas as pl
from jax.experimental.pallas import tpu as pltpu
from jax.experimental.pallas import tpu_sc as plsc

EPS = 1e-8

NC = 2
NS = 16
LANES = 16

CHUNK = 80
ROW_BLOCK = 1000


def _matmul_body(x_ref, w_ref, b_ref, out_ref):
    x = x_ref[...]
    w = w_ref[...]
    p = lax.dot_general(x, w, (((1,), (1,)), ((), ())),
                        preferred_element_type=jnp.float32)
    p = p + b_ref[0:1, :]
    t = out_ref.shape[0]
    d = out_ref.shape[2]
    for i in range(t):
        out_ref[i] = p[:, i * d:(i + 1) * d]


def _transform(node_states, W, b):
    n, d = node_states.shape
    t = W.shape[0] // d
    bb = jnp.broadcast_to(b.reshape(1, -1), (8, t * d))
    grid = n // ROW_BLOCK
    return pl.pallas_call(
        _matmul_body,
        grid=(grid,),
        in_specs=[
            pl.BlockSpec((ROW_BLOCK, d), lambda i: (i, 0)),
            pl.BlockSpec((t * d, d), lambda i: (0, 0)),
            pl.BlockSpec((8, t * d), lambda i: (0, 0)),
        ],
        out_specs=pl.BlockSpec((t, ROW_BLOCK, d), lambda i: (0, i, 0)),
        out_shape=jax.ShapeDtypeStruct((t, n, d), jnp.float32),
    )(node_states, W, bb)


def _sc_body(n_nodes, dim, steps, table, src, tgt, acc_out, cnt_out,
             src_buf, tgt_buf, rows_a, rows_b, ones_v, acc_sh, cnt_sh,
             sem_a, sem_b):
    c = lax.axis_index("c")
    s = lax.axis_index("s")
    rows_per_tile = n_nodes // NS

    pltpu.sync_copy(src.at[c, s], src_buf)
    pltpu.sync_copy(tgt.at[c, s], tgt_buf)

    zero16 = jnp.zeros((LANES,), jnp.float32)
    offv = jnp.full((LANES,), c * n_nodes, jnp.int32)

    def add_off(g, _):
        def inner(l, _):
            sl = pl.ds(l * LANES, LANES)
            src_buf[g, sl] = src_buf[g, sl] + offv
            return 0
        return lax.fori_loop(0, CHUNK // LANES, inner, 0)
    lax.fori_loop(0, steps, add_off, 0)

    def zrow(i, _):
        def zl(l, _):
            rows_a[i, pl.ds(l * LANES, LANES)] = zero16
            return 0
        return lax.fori_loop(0, dim // LANES, zl, 0)
    lax.fori_loop(0, CHUNK, zrow, 0)

    def zcnt(i, _):
        ones_v[i] = zero16
        return 0
    lax.fori_loop(0, CHUNK, zcnt, 0)

    base = s * rows_per_tile
    full, rem = divmod(rows_per_tile, CHUNK)
    for r in range(full):
        pltpu.sync_copy(rows_a, acc_sh.at[pl.ds(base + r * CHUNK, CHUNK)])
        pltpu.sync_copy(ones_v, cnt_sh.at[pl.ds(base + r * CHUNK, CHUNK)])
    if rem:
        pltpu.sync_copy(rows_a.at[pl.ds(0, rem)],
                        acc_sh.at[pl.ds(base + full * CHUNK, rem)])
        pltpu.sync_copy(ones_v.at[pl.ds(0, rem)],
                        cnt_sh.at[pl.ds(base + full * CHUNK, rem)])

    one16 = jnp.ones((LANES,), jnp.float32)

    def frow(i, _):
        ones_v[i] = one16
        return 0
    lax.fori_loop(0, CHUNK, frow, 0)

    plsc.subcore_barrier()

    def gather(j, rows, sem):
        return pltpu.async_copy(table.at[src_buf.at[j]], rows, sem)

    def gather_wait(j, rows, sem):
        pltpu.make_async_copy(table.at[src_buf.at[j]], rows, sem).wait()

    def scatter(j, rows):
        pltpu.sync_copy(rows, acc_sh.at[tgt_buf.at[j]], add=True)
        pltpu.sync_copy(ones_v, cnt_sh.at[tgt_buf.at[j]], add=True)

    gather(0, rows_a, sem_a)

    def pair(p, _):
        jj = 2 * p
        gather_wait(jj, rows_a, sem_a)
        gather(jj + 1, rows_b, sem_b)
        scatter(jj, rows_a)
        gather_wait(jj + 1, rows_b, sem_b)
        gather(jj + 2, rows_a, sem_a)
        scatter(jj + 1, rows_b)
        return 0

    lax.fori_loop(0, (steps - 1) // 2, pair, 0)
    gather_wait(steps - 1, rows_a, sem_a)
    scatter(steps - 1, rows_a)

    plsc.subcore_barrier()

    pltpu.sync_copy(acc_sh.at[pl.ds(base, rows_per_tile)],
                    acc_out.at[c, pl.ds(base, rows_per_tile)])
    pltpu.sync_copy(cnt_sh.at[pl.ds(base, rows_per_tile)],
                    cnt_out.at[c, pl.ds(base, rows_per_tile)])


def _sc_scatter(table, src, tgt, n_nodes, dim, steps):
    mesh = plsc.VectorSubcoreMesh(core_axis_name="c", subcore_axis_name="s",
                                  num_cores=NC, num_subcores=NS)
    fn = pl.kernel(
        functools.partial(_sc_body, n_nodes, dim, steps),
        out_type=(
            jax.ShapeDtypeStruct((NC, n_nodes, dim), jnp.float32),
            jax.ShapeDtypeStruct((NC, n_nodes, LANES), jnp.float32),
        ),
        mesh=mesh,
        scratch_types=(
            pltpu.VMEM((steps, CHUNK), jnp.int32),
            pltpu.VMEM((steps, CHUNK), jnp.int32),
            pltpu.VMEM((CHUNK, dim), jnp.float32),
            pltpu.VMEM((CHUNK, dim), jnp.float32),
            pltpu.VMEM((CHUNK, LANES), jnp.float32),
            pltpu.VMEM_SHARED((n_nodes, dim), jnp.float32),
            pltpu.VMEM_SHARED((n_nodes, LANES), jnp.float32),
            pltpu.SemaphoreType.DMA,
            pltpu.SemaphoreType.DMA,
        ),
    )
    return fn(table, src, tgt)


def _combine_body(acc_ref, cnt_ref, out_ref):
    ssum = acc_ref[0] + acc_ref[1]
    cc = cnt_ref[0, :, 0:1] + cnt_ref[1, :, 0:1]
    div = jnp.where(cc == 0.0, 1.0, cc)
    out_ref[...] = ssum / div + EPS


def _combine(acc, cnt):
    _, n, d = acc.shape
    grid = n // ROW_BLOCK
    return pl.pallas_call(
        _combine_body,
        grid=(grid,),
        in_specs=[
            pl.BlockSpec((NC, ROW_BLOCK, d), lambda i: (0, i, 0)),
            pl.BlockSpec((NC, ROW_BLOCK, LANES), lambda i: (0, i, 0)),
        ],
        out_specs=pl.BlockSpec((ROW_BLOCK, d), lambda i: (i, 0)),
        out_shape=jax.ShapeDtypeStruct((n, d), jnp.float32),
    )(acc, cnt)


@jax.jit
def kernel(edge_lists, node_states, W, b):
    t, m, _ = edge_lists.shape
    n_nodes, dim = node_states.shape
    edges_per_tile = m // NS
    steps = edges_per_tile // CHUNK

    el = edge_lists.astype(jnp.int32)
    src = el[..., 0].reshape(t, NS, steps, CHUNK)
    tgt = el[..., 1].reshape(t, NS, steps, CHUNK)

    table = _transform(node_states, W, b).reshape(t * n_nodes, dim)
    acc, cnt = _sc_scatter(table, src, tgt, n_nodes, dim, steps)
    return _combine(acc, cnt)


# speedup vs baseline: 5.4604x; 5.4604x over previous
import functools

import jax
import jax.numpy as jnp
from jax import lax
from jax.experimental import pallas as pl
from jax.experimental.pallas import tpu as pltpu
from jax.experimental.pallas import tpu_sc as plsc

EPS = 1e-8

NC = 2
NS = 16
LANES = 16

CHUNK = 80
ROW_BLOCK = 1000


def _matmul_body(x_ref, w_ref, b_ref, out_ref):
    x = x_ref[...]
    w = w_ref[...]
    p = lax.dot_general(x, w, (((1,), (1,)), ((), ())),
                        preferred_element_type=jnp.float32)
    p = p + b_ref[0:1, :]
    t = out_ref.shape[1]
    dh = out_ref.shape[3]
    for h in range(NC):
        for i in range(t):
            out_ref[h, i] = p[:, i * NC * dh + h * dh:(i * NC + h + 1) * dh]


def _transform(node_states, W, b):
    n, d = node_states.shape
    t = W.shape[0] // d
    dh = d // NC
    bb = jnp.broadcast_to(b.reshape(1, -1), (8, t * d))
    grid = n // ROW_BLOCK
    return pl.pallas_call(
        _matmul_body,
        grid=(grid,),
        in_specs=[
            pl.BlockSpec((ROW_BLOCK, d), lambda i: (i, 0)),
            pl.BlockSpec((t * d, d), lambda i: (0, 0)),
            pl.BlockSpec((8, t * d), lambda i: (0, 0)),
        ],
        out_specs=pl.BlockSpec((NC, t, ROW_BLOCK, dh), lambda i: (0, 0, i, 0)),
        out_shape=jax.ShapeDtypeStruct((NC, t, n, dh), jnp.float32),
    )(node_states, W, bb)


def _sc_body(n_nodes, n_pad, dh, steps, tiles_per_type,
             table, src, tgt, acc_out, cnt_out,
             src_buf, tgt_buf, rows_a, rows_b, ones_v, acc_sh, cnt_sh,
             sem_a, sem_b):
    c = lax.axis_index("c")
    s = lax.axis_index("s")
    rows_per_tile = n_pad // NS

    pltpu.sync_copy(src.at[s], src_buf)
    pltpu.sync_copy(tgt.at[s], tgt_buf)

    zero16 = jnp.zeros((LANES,), jnp.float32)
    type_off = jnp.where(s >= tiles_per_type, n_nodes, 0)
    off = (c * (NC * n_nodes) + type_off).astype(jnp.int32)
    offv = jnp.full((LANES,), 1, jnp.int32) * off

    def add_off(g, _):
        def inner(l, _):
            sl = pl.ds(l * LANES, LANES)
            src_buf[g, sl] = src_buf[g, sl] + offv
            return 0
        return lax.fori_loop(0, CHUNK // LANES, inner, 0)
    lax.fori_loop(0, steps, add_off, 0)

    def zrow(i, _):
        def zl(l, _):
            rows_a[i, pl.ds(l * LANES, LANES)] = zero16
            return 0
        return lax.fori_loop(0, dh // LANES, zl, 0)
    lax.fori_loop(0, CHUNK, zrow, 0)

    def zcnt(i, _):
        ones_v[i] = zero16
        return 0
    lax.fori_loop(0, CHUNK, zcnt, 0)

    base = s * rows_per_tile
    full, rem = divmod(rows_per_tile, CHUNK)
    for r in range(full):
        pltpu.sync_copy(rows_a, acc_sh.at[pl.ds(base + r * CHUNK, CHUNK)])
        pltpu.sync_copy(ones_v, cnt_sh.at[pl.ds(base + r * CHUNK, CHUNK)])
    if rem:
        pltpu.sync_copy(rows_a.at[pl.ds(0, rem)],
                        acc_sh.at[pl.ds(base + full * CHUNK, rem)])
        pltpu.sync_copy(ones_v.at[pl.ds(0, rem)],
                        cnt_sh.at[pl.ds(base + full * CHUNK, rem)])

    one16 = jnp.ones((LANES,), jnp.float32)

    def frow(i, _):
        ones_v[i] = one16
        return 0
    lax.fori_loop(0, CHUNK, frow, 0)

    plsc.subcore_barrier()

    def gather(j, rows, sem):
        return pltpu.async_copy(table.at[src_buf.at[j]], rows, sem)

    def gather_wait(j, rows, sem):
        pltpu.make_async_copy(table.at[src_buf.at[j]], rows, sem).wait()

    def scatter(j, rows, parity):
        pltpu.sync_copy(rows, acc_sh.at[tgt_buf.at[j]], add=True)

        @pl.when(c == parity)
        def _():
            pltpu.sync_copy(ones_v, cnt_sh.at[tgt_buf.at[j]], add=True)

    gather(0, rows_a, sem_a)

    def pair(p, _):
        jj = 2 * p
        gather_wait(jj, rows_a, sem_a)
        gather(jj + 1, rows_b, sem_b)
        scatter(jj, rows_a, 0)
        gather_wait(jj + 1, rows_b, sem_b)

        @pl.when(jj + 2 < steps)
        def _():
            gather(jj + 2, rows_a, sem_a)

        scatter(jj + 1, rows_b, 1)
        return 0

    lax.fori_loop(0, steps // 2, pair, 0)

    plsc.subcore_barrier()

    pltpu.sync_copy(acc_sh.at[pl.ds(base, rows_per_tile)],
                    acc_out.at[c, pl.ds(base, rows_per_tile)])
    pltpu.sync_copy(cnt_sh.at[pl.ds(base, rows_per_tile)],
                    cnt_out.at[c, pl.ds(base, rows_per_tile)])


def _sc_scatter(table, src, tgt, n_nodes, n_pad, dh, steps, tiles_per_type):
    mesh = plsc.VectorSubcoreMesh(core_axis_name="c", subcore_axis_name="s",
                                  num_cores=NC, num_subcores=NS)
    fn = pl.kernel(
        functools.partial(_sc_body, n_nodes, n_pad, dh, steps,
                          tiles_per_type),
        out_type=(
            jax.ShapeDtypeStruct((NC, n_pad, dh), jnp.float32),
            jax.ShapeDtypeStruct((NC, n_pad, LANES), jnp.float32),
        ),
        mesh=mesh,
        scratch_types=(
            pltpu.VMEM((steps, CHUNK), jnp.int32),
            pltpu.VMEM((steps, CHUNK), jnp.int32),
            pltpu.VMEM((CHUNK, dh), jnp.float32),
            pltpu.VMEM((CHUNK, dh), jnp.float32),
            pltpu.VMEM((CHUNK, LANES), jnp.float32),
            pltpu.VMEM_SHARED((n_pad, dh), jnp.float32),
            pltpu.VMEM_SHARED((n_pad, LANES), jnp.float32),
            pltpu.SemaphoreType.DMA,
            pltpu.SemaphoreType.DMA,
        ),
        compiler_params=pltpu.CompilerParams(use_tc_tiling_on_sc=False),
    )
    return fn(table, src, tgt)


def _combine_body(acc_ref, cnt_ref, out_ref):
    cc = cnt_ref[0, :, 0:1] + cnt_ref[1, :, 0:1]
    div = jnp.where(cc == 0.0, 1.0, cc)
    dh = acc_ref.shape[2]
    for h in range(NC):
        out_ref[:, h * dh:(h + 1) * dh] = acc_ref[h] / div + EPS


def _combine(acc, cnt, n):
    dh = acc.shape[2]
    grid = n // ROW_BLOCK
    return pl.pallas_call(
        _combine_body,
        grid=(grid,),
        in_specs=[
            pl.BlockSpec((NC, ROW_BLOCK, dh), lambda i: (0, i, 0)),
            pl.BlockSpec((NC, ROW_BLOCK, LANES), lambda i: (0, i, 0)),
        ],
        out_specs=pl.BlockSpec((ROW_BLOCK, NC * dh), lambda i: (i, 0)),
        out_shape=jax.ShapeDtypeStruct((n, NC * dh), jnp.float32),
    )(acc, cnt)


@jax.jit
def kernel(edge_lists, node_states, W, b):
    t, m, _ = edge_lists.shape
    n_nodes, dim = node_states.shape
    dh = dim // NC
    edges_per_tile = t * m // NS
    steps = edges_per_tile // CHUNK
    tiles_per_type = NS // t

    el = edge_lists.astype(jnp.int32)
    src = el[..., 0].reshape(NS, steps, CHUNK)
    tgt = el[..., 1].reshape(NS, steps, CHUNK)

    n_pad = ((n_nodes + NS * 8 - 1) // (NS * 8)) * (NS * 8)

    table = _transform(node_states, W, b).reshape(NC * t * n_nodes, dh)
    acc, cnt = _sc_scatter(table, src, tgt, n_nodes, n_pad, dh, steps,
                           tiles_per_type)
    return _combine(acc, cnt, n_nodes)
